# baseline (speedup 1.0000x reference)
import functools

import jax
import jax.numpy as jnp
from jax import lax
from jax.experimental import pallas as pl
from jax.experimental.pallas import tpu as pltpu
from jax.experimental.pallas import tpu_sc as plsc


T = 8192
D_IN = 2048
D_OUT = 2048
MAX_LORAS = 8
RANK = 16
LR = MAX_LORAS * RANK

BM = 512
CH = 32


def _sc_bias_gather(lora_bias, idx):
    info = plsc.get_sparse_core_info()
    nw = info.num_cores * info.num_subcores
    b_per_w = T // nw
    mesh = plsc.VectorSubcoreMesh(core_axis_name="c", subcore_axis_name="s")

    @functools.partial(
        pl.kernel, mesh=mesh,
        out_type=jax.ShapeDtypeStruct((T, D_OUT), jnp.float32),
        scratch_types=[
            pltpu.VMEM((CH,), jnp.int32),
            pltpu.VMEM((CH, D_OUT), jnp.float32),
            pltpu.SemaphoreType.DMA,
        ],
    )
    def k(bias_hbm, idx_hbm, out_hbm, idx_v, rows_v, sem):
        wid = lax.axis_index("s") * info.num_cores + lax.axis_index("c")
        base = wid * b_per_w

        def body(j, carry):
            off = base + j * CH
            pltpu.sync_copy(idx_hbm.at[pl.ds(off, CH)], idx_v)
            pltpu.async_copy(bias_hbm.at[idx_v], rows_v, sem).wait()
            pltpu.sync_copy(rows_v, out_hbm.at[pl.ds(off, CH)])
            return carry

        lax.fori_loop(0, b_per_w // CH, body, 0)

    return k(lora_bias, idx)


def _fused_body(idx_ref, x_ref, w_ref, a_ref, b_ref, bias_ref, out_ref):
    x = x_ref[...]
    base = jnp.dot(x, w_ref[...], preferred_element_type=jnp.float32)
    shrink = jax.lax.dot_general(
        x, a_ref[...], (((1,), (1,)), ((), ())),
        preferred_element_type=jnp.float32)
    idx = idx_ref[...]
    grp = jax.lax.broadcasted_iota(jnp.int32, (BM, LR), 1) // RANK
    mshrink = jnp.where(grp == idx, shrink, 0.0)
    expand = jnp.dot(mshrink, b_ref[...], preferred_element_type=jnp.float32)
    out_ref[...] = base + expand + bias_ref[...]


@jax.jit
def kernel(x, token_lora_indices, W, lora_a, lora_b, lora_bias):
    idx32 = token_lora_indices.astype(jnp.int32)
    idx = idx32.reshape(T, 1)
    a_cat = lora_a.reshape(LR, D_IN)
    b_cat = jnp.transpose(lora_b, (0, 2, 1)).reshape(LR, D_OUT)
    bias_full = _sc_bias_gather(lora_bias, idx32)

    grid = (T // BM,)
    return pl.pallas_call(
        _fused_body,
        grid=grid,
        in_specs=[
            pl.BlockSpec((BM, 1), lambda i: (i, 0)),
            pl.BlockSpec((BM, D_IN), lambda i: (i, 0)),
            pl.BlockSpec((D_IN, D_OUT), lambda i: (0, 0)),
            pl.BlockSpec((LR, D_IN), lambda i: (0, 0)),
            pl.BlockSpec((LR, D_OUT), lambda i: (0, 0)),
            pl.BlockSpec((BM, D_OUT), lambda i: (i, 0)),
        ],
        out_specs=pl.BlockSpec((BM, D_OUT), lambda i: (i, 0)),
        out_shape=jax.ShapeDtypeStruct((T, D_OUT), jnp.float32),
    )(idx, x, W, a_cat, b_cat, bias_full)


# speedup vs baseline: 2.5926x; 2.5926x over previous
import functools

import jax
import jax.numpy as jnp
from jax.experimental import pallas as pl


T = 8192
D_IN = 2048
D_OUT = 2048
MAX_LORAS = 8
RANK = 16
LR = MAX_LORAS * RANK

BM = 512


def _fused_body(idx_ref, x_ref, w_ref, a_ref, b_ref, bias_ref, out_ref):
    x = x_ref[...]
    base = jnp.dot(x, w_ref[...], preferred_element_type=jnp.float32)
    shrink = jnp.dot(x, a_ref[...], preferred_element_type=jnp.float32)
    idx = idx_ref[...]
    grp = jax.lax.broadcasted_iota(jnp.int32, (BM, LR), 1) // RANK
    mshrink = jnp.where(grp == idx, shrink, 0.0)
    expand = jnp.dot(mshrink, b_ref[...], preferred_element_type=jnp.float32)
    lane8 = jax.lax.broadcasted_iota(jnp.int32, (BM, MAX_LORAS), 1)
    onehot = (lane8 == idx).astype(jnp.float32)
    bias = jnp.dot(onehot, bias_ref[...], preferred_element_type=jnp.float32)
    out_ref[...] = base + expand + bias


@jax.jit
def kernel(x, token_lora_indices, W, lora_a, lora_b, lora_bias):
    idx = token_lora_indices.astype(jnp.int32).reshape(T, 1)
    a_cat = lora_a.reshape(LR, D_IN).T
    b_cat = jnp.transpose(lora_b, (0, 2, 1)).reshape(LR, D_OUT)

    grid = (T // BM,)
    return pl.pallas_call(
        _fused_body,
        grid=grid,
        in_specs=[
            pl.BlockSpec((BM, 1), lambda i: (i, 0)),
            pl.BlockSpec((BM, D_IN), lambda i: (i, 0)),
            pl.BlockSpec((D_IN, D_OUT), lambda i: (0, 0)),
            pl.BlockSpec((D_IN, LR), lambda i: (0, 0)),
            pl.BlockSpec((LR, D_OUT), lambda i: (0, 0)),
            pl.BlockSpec((MAX_LORAS, D_OUT), lambda i: (0, 0)),
        ],
        out_specs=pl.BlockSpec((BM, D_OUT), lambda i: (i, 0)),
        out_shape=jax.ShapeDtypeStruct((T, D_OUT), jnp.float32),
    )(idx, x, W, a_cat, b_cat, lora_bias)
